# baseline (speedup 1.0000x reference)
import jax
import jax.numpy as jnp
from jax import lax
from jax.experimental import pallas as pl
from jax.experimental.pallas import tpu as pltpu
from jax.experimental.pallas import tpu_sc as plsc

NC, NS, L = 2, 16, 16
NW = NC * NS
B, T, N = 32, 12, 1024
BT = B * T
DAY_SIZE, TIME_SIZE, NODE_SIZE = 32, 32, 64
C = DAY_SIZE + TIME_SIZE + NODE_SIZE
BPW = BT // NW


def _sc_gather_body(didx_hbm, tidx_hbm, wday_hbm, wtime_hbm,
                    drows_hbm, trows_hbm,
                    didx_v, tidx_v, drow_v, trow_v, sem):
    cid = lax.axis_index("c")
    sid = lax.axis_index("s")
    wid = sid * NC + cid

    pltpu.sync_copy(didx_hbm.at[wid], didx_v)
    pltpu.sync_copy(tidx_hbm.at[wid], tidx_v)
    pltpu.async_copy(wday_hbm.at[didx_v], drow_v, sem).wait()
    pltpu.async_copy(wtime_hbm.at[tidx_v], trow_v, sem).wait()
    pltpu.sync_copy(drow_v, drows_hbm.at[pl.ds(wid * BPW, BPW), :])
    pltpu.sync_copy(trow_v, trows_hbm.at[pl.ds(wid * BPW, BPW), :])


def _tc_broadcast_body(d_ref, t_ref, node_ref, out_ref):
    out_ref[0, :, 0:DAY_SIZE] = jnp.broadcast_to(d_ref[0], (N, DAY_SIZE))
    out_ref[0, :, DAY_SIZE:DAY_SIZE + TIME_SIZE] = jnp.broadcast_to(
        t_ref[0], (N, TIME_SIZE))
    out_ref[0, :, DAY_SIZE + TIME_SIZE:C] = node_ref[...]


def kernel(daytime, W_day, W_time, W_node):
    dt = daytime.astype(jnp.int32)
    day_idx = dt[..., 0].reshape(NW, BPW)
    time_idx = dt[..., 1].reshape(NW, BPW)

    mesh = plsc.VectorSubcoreMesh(core_axis_name="c", subcore_axis_name="s",
                                  num_cores=NC, num_subcores=NS)
    drows, trows = pl.kernel(
        _sc_gather_body,
        out_type=(jax.ShapeDtypeStruct((BT, DAY_SIZE), jnp.float32),
                  jax.ShapeDtypeStruct((BT, TIME_SIZE), jnp.float32)),
        mesh=mesh,
        compiler_params=pltpu.CompilerParams(use_tc_tiling_on_sc=False),
        scratch_types=[
            pltpu.VMEM((BPW,), jnp.int32),
            pltpu.VMEM((BPW,), jnp.int32),
            pltpu.VMEM((BPW, DAY_SIZE), jnp.float32),
            pltpu.VMEM((BPW, TIME_SIZE), jnp.float32),
            pltpu.SemaphoreType.DMA,
        ],
    )(day_idx, time_idx, W_day, W_time)

    out = pl.pallas_call(
        _tc_broadcast_body,
        grid=(BT,),
        in_specs=[
            pl.BlockSpec((1, 1, DAY_SIZE), lambda i: (i, 0, 0)),
            pl.BlockSpec((1, 1, TIME_SIZE), lambda i: (i, 0, 0)),
            pl.BlockSpec((N, NODE_SIZE), lambda i: (0, 0)),
        ],
        out_specs=pl.BlockSpec((1, N, C), lambda i: (i, 0, 0)),
        out_shape=jax.ShapeDtypeStruct((BT, N, C), jnp.float32),
    )(drows.reshape(BT, 1, DAY_SIZE), trows.reshape(BT, 1, TIME_SIZE), W_node)
    return out.reshape(B, T, N, C)


# speedup vs baseline: 1.0673x; 1.0673x over previous
import jax
import jax.numpy as jnp
from jax import lax
from jax.experimental import pallas as pl
from jax.experimental.pallas import tpu as pltpu
from jax.experimental.pallas import tpu_sc as plsc

NC, NS, L = 2, 16, 16
NW = NC * NS
B, T, N = 32, 12, 1024
BT = B * T
DAY_SIZE, TIME_SIZE, NODE_SIZE = 32, 32, 64
C = DAY_SIZE + TIME_SIZE + NODE_SIZE
NPW = N // NW
IDX_CHUNK = 128
N_CHUNKS = BT // IDX_CHUNK
K = 4
NBUF = 4


def _sc_body(didx_hbm, tidx_hbm, wday_hbm, wtime_hbm, wnode_hbm, out_hbm,
             didx_v, tidx_v, drows_v, trows_v, node_v,
             buf0, buf1, buf2, buf3, sem_g, sem0, sem1, sem2, sem3):
    bufs = (buf0, buf1, buf2, buf3)
    sems = (sem0, sem1, sem2, sem3)
    cid = lax.axis_index("c")
    sid = lax.axis_index("s")
    wid = sid * NC + cid
    n0 = wid * NPW

    pltpu.sync_copy(didx_hbm, didx_v)
    pltpu.sync_copy(tidx_hbm, tidx_v)

    for j in range(N_CHUNKS):
        pltpu.async_copy(
            wday_hbm.at[didx_v.at[j]],
            drows_v.at[pl.ds(j * IDX_CHUNK, IDX_CHUNK), :], sem_g).wait()
        pltpu.async_copy(
            wtime_hbm.at[tidx_v.at[j]],
            trows_v.at[pl.ds(j * IDX_CHUNK, IDX_CHUNK), :], sem_g).wait()

    pltpu.sync_copy(wnode_hbm.at[pl.ds(n0, NPW)], node_v)
    for buf in bufs:
        def init_k(k, carry):
            for r in range(NPW):
                for j in range(NODE_SIZE // L):
                    buf[k, r, pl.ds(NODE_SIZE + j * L, L)] = \
                        node_v[r, pl.ds(j * L, L)]
            return carry
        lax.fori_loop(0, K, init_k, 0)

    def fill_and_send(bt0, buf, sem):
        def fill_k(k, carry):
            bt = bt0 + k
            d0 = drows_v[bt, pl.ds(0, L)]
            d1 = drows_v[bt, pl.ds(L, L)]
            t0 = trows_v[bt, pl.ds(0, L)]
            t1 = trows_v[bt, pl.ds(L, L)]
            for r in range(NPW):
                buf[k, r, pl.ds(0, L)] = d0
                buf[k, r, pl.ds(L, L)] = d1
                buf[k, r, pl.ds(2 * L, L)] = t0
                buf[k, r, pl.ds(3 * L, L)] = t1
            return carry
        if True:
            lax.fori_loop(0, K, fill_k, 0)
        pltpu.async_copy(buf, out_hbm.at[pl.ds(bt0, K), pl.ds(n0, NPW), :], sem)

    def wait_prev(buf, sem):
        pltpu.make_async_copy(
            buf, out_hbm.at[pl.ds(0, K), pl.ds(n0, NPW), :], sem).wait()

    for j in range(NBUF):
        fill_and_send(j * K, bufs[j], sems[j])

    def body(i, carry):
        bt0 = i * NBUF * K
        for j in range(NBUF):
            wait_prev(bufs[j], sems[j])
            fill_and_send(bt0 + j * K, bufs[j], sems[j])
        return carry

    lax.fori_loop(1, BT // (NBUF * K), body, 0)
    for j in range(NBUF):
        wait_prev(bufs[j], sems[j])


def kernel(daytime, W_day, W_time, W_node):
    dt = daytime.astype(jnp.int32)
    day_idx = dt[..., 0].reshape(N_CHUNKS, IDX_CHUNK)
    time_idx = dt[..., 1].reshape(N_CHUNKS, IDX_CHUNK)

    mesh = plsc.VectorSubcoreMesh(core_axis_name="c", subcore_axis_name="s",
                                  num_cores=NC, num_subcores=NS)
    out = pl.kernel(
        _sc_body,
        out_type=jax.ShapeDtypeStruct((BT, N, C), jnp.float32),
        mesh=mesh,
        compiler_params=pltpu.CompilerParams(use_tc_tiling_on_sc=False),
        scratch_types=[
            pltpu.VMEM((N_CHUNKS, IDX_CHUNK), jnp.int32),
            pltpu.VMEM((N_CHUNKS, IDX_CHUNK), jnp.int32),
            pltpu.VMEM((BT, DAY_SIZE), jnp.float32),
            pltpu.VMEM((BT, TIME_SIZE), jnp.float32),
            pltpu.VMEM((NPW, NODE_SIZE), jnp.float32),
            pltpu.VMEM((K, NPW, C), jnp.float32),
            pltpu.VMEM((K, NPW, C), jnp.float32),
            pltpu.VMEM((K, NPW, C), jnp.float32),
            pltpu.VMEM((K, NPW, C), jnp.float32),
            pltpu.SemaphoreType.DMA,
            pltpu.SemaphoreType.DMA,
            pltpu.SemaphoreType.DMA,
            pltpu.SemaphoreType.DMA,
            pltpu.SemaphoreType.DMA,
        ],
    )(day_idx, time_idx, W_day, W_time, W_node)
    return out.reshape(B, T, N, C)


# speedup vs baseline: 1.1181x; 1.0476x over previous
import jax
import jax.numpy as jnp
from jax import lax
from jax.experimental import pallas as pl
from jax.experimental.pallas import tpu as pltpu
from jax.experimental.pallas import tpu_sc as plsc

NC, NS, L = 2, 16, 16
NW = NC * NS
B, T, N = 32, 12, 1024
BT = B * T
DAY_SIZE, TIME_SIZE, NODE_SIZE = 32, 32, 64
C = DAY_SIZE + TIME_SIZE + NODE_SIZE
NPW = N // NW
IDX_CHUNK = 128
N_CHUNKS = BT // IDX_CHUNK
K = 4
NBUF = 4


def _sc_body(didx_hbm, tidx_hbm, wday_hbm, wtime_hbm, wnode_hbm, out_hbm,
             didx_v, tidx_v, drows_v, trows_v, node_v,
             buf0, buf1, buf2, buf3, sem_g, sem0, sem1, sem2, sem3):
    bufs = (buf0, buf1, buf2, buf3)
    sems = (sem0, sem1, sem2, sem3)
    cid = lax.axis_index("c")
    sid = lax.axis_index("s")
    wid = sid * NC + cid
    n0 = wid * NPW

    pltpu.sync_copy(didx_hbm, didx_v)
    pltpu.sync_copy(tidx_hbm, tidx_v)

    gathers = []
    for j in range(N_CHUNKS):
        gathers.append(pltpu.async_copy(
            wday_hbm.at[didx_v.at[j]],
            drows_v.at[pl.ds(j * IDX_CHUNK, IDX_CHUNK), :], sem_g))
        gathers.append(pltpu.async_copy(
            wtime_hbm.at[tidx_v.at[j]],
            trows_v.at[pl.ds(j * IDX_CHUNK, IDX_CHUNK), :], sem_g))

    pltpu.sync_copy(wnode_hbm.at[pl.ds(n0, NPW)], node_v)
    for buf in bufs:
        def init_k(k, carry):
            for r in range(NPW):
                for j in range(NODE_SIZE // L):
                    buf[k, r, pl.ds(NODE_SIZE + j * L, L)] = \
                        node_v[r, pl.ds(j * L, L)]
            return carry
        lax.fori_loop(0, K, init_k, 0)

    def fill_and_send(bt0, buf, sem):
        def fill_k(k, carry):
            bt = bt0 + k
            d0 = drows_v[bt, pl.ds(0, L)]
            d1 = drows_v[bt, pl.ds(L, L)]
            t0 = trows_v[bt, pl.ds(0, L)]
            t1 = trows_v[bt, pl.ds(L, L)]
            for r in range(NPW):
                buf[k, r, pl.ds(0, L)] = d0
                buf[k, r, pl.ds(L, L)] = d1
                buf[k, r, pl.ds(2 * L, L)] = t0
                buf[k, r, pl.ds(3 * L, L)] = t1
            return carry
        if True:
            lax.fori_loop(0, K, fill_k, 0)
        pltpu.async_copy(buf, out_hbm.at[pl.ds(bt0, K), pl.ds(n0, NPW), :], sem)

    def wait_prev(buf, sem):
        pltpu.make_async_copy(
            buf, out_hbm.at[pl.ds(0, K), pl.ds(n0, NPW), :], sem).wait()

    for g in gathers:
        g.wait()

    for j in range(NBUF):
        fill_and_send(j * K, bufs[j], sems[j])

    def body(i, carry):
        bt0 = i * NBUF * K
        for j in range(NBUF):
            wait_prev(bufs[j], sems[j])
            fill_and_send(bt0 + j * K, bufs[j], sems[j])
        return carry

    lax.fori_loop(1, BT // (NBUF * K), body, 0)
    for j in range(NBUF):
        wait_prev(bufs[j], sems[j])


def kernel(daytime, W_day, W_time, W_node):
    dt = daytime.astype(jnp.int32)
    day_idx = dt[..., 0].reshape(N_CHUNKS, IDX_CHUNK)
    time_idx = dt[..., 1].reshape(N_CHUNKS, IDX_CHUNK)

    mesh = plsc.VectorSubcoreMesh(core_axis_name="c", subcore_axis_name="s",
                                  num_cores=NC, num_subcores=NS)
    out = pl.kernel(
        _sc_body,
        out_type=jax.ShapeDtypeStruct((BT, N, C), jnp.float32),
        mesh=mesh,
        compiler_params=pltpu.CompilerParams(use_tc_tiling_on_sc=False),
        scratch_types=[
            pltpu.VMEM((N_CHUNKS, IDX_CHUNK), jnp.int32),
            pltpu.VMEM((N_CHUNKS, IDX_CHUNK), jnp.int32),
            pltpu.VMEM((BT, DAY_SIZE), jnp.float32),
            pltpu.VMEM((BT, TIME_SIZE), jnp.float32),
            pltpu.VMEM((NPW, NODE_SIZE), jnp.float32),
            pltpu.VMEM((K, NPW, C), jnp.float32),
            pltpu.VMEM((K, NPW, C), jnp.float32),
            pltpu.VMEM((K, NPW, C), jnp.float32),
            pltpu.VMEM((K, NPW, C), jnp.float32),
            pltpu.SemaphoreType.DMA,
            pltpu.SemaphoreType.DMA,
            pltpu.SemaphoreType.DMA,
            pltpu.SemaphoreType.DMA,
            pltpu.SemaphoreType.DMA,
        ],
    )(day_idx, time_idx, W_day, W_time, W_node)
    return out.reshape(B, T, N, C)


# speedup vs baseline: 1.2191x; 1.0903x over previous
import jax
import jax.numpy as jnp
from jax import lax
from jax.experimental import pallas as pl
from jax.experimental.pallas import tpu as pltpu
from jax.experimental.pallas import tpu_sc as plsc

NC, NS, L = 2, 16, 16
NW = NC * NS
B, T, N = 32, 12, 1024
BT = B * T
DAY_SIZE, TIME_SIZE, NODE_SIZE = 32, 32, 64
C = DAY_SIZE + TIME_SIZE + NODE_SIZE
NPW = N // NW
IDX_CHUNK = 128
N_CHUNKS = BT // IDX_CHUNK
K = 4
NBUF = 4


def _sc_body(idx_hbm, wday_hbm, wtime_hbm, wnode_hbm, out_hbm,
             idx_v, drows_v, trows_v, node_v,
             buf0, buf1, buf2, buf3, sem_g, sem0, sem1, sem2, sem3):
    bufs = (buf0, buf1, buf2, buf3)
    sems = (sem0, sem1, sem2, sem3)
    cid = lax.axis_index("c")
    sid = lax.axis_index("s")
    wid = sid * NC + cid
    n0 = wid * NPW

    pltpu.sync_copy(idx_hbm, idx_v)

    gathers = []
    for j in range(N_CHUNKS):
        gathers.append(pltpu.async_copy(
            wday_hbm.at[idx_v.at[j]],
            drows_v.at[pl.ds(j * IDX_CHUNK, IDX_CHUNK), :], sem_g))
        gathers.append(pltpu.async_copy(
            wtime_hbm.at[idx_v.at[N_CHUNKS + j]],
            trows_v.at[pl.ds(j * IDX_CHUNK, IDX_CHUNK), :], sem_g))

    pltpu.sync_copy(wnode_hbm.at[pl.ds(n0, NPW)], node_v)
    for buf in bufs:
        def init_k(k, carry):
            for r in range(NPW):
                for j in range(NODE_SIZE // L):
                    buf[k, r, pl.ds(NODE_SIZE + j * L, L)] = \
                        node_v[r, pl.ds(j * L, L)]
            return carry
        lax.fori_loop(0, K, init_k, 0)

    def fill_and_send(bt0, buf, sem):
        def fill_k(k, carry):
            bt = bt0 + k
            d0 = drows_v[bt, pl.ds(0, L)]
            d1 = drows_v[bt, pl.ds(L, L)]
            t0 = trows_v[bt, pl.ds(0, L)]
            t1 = trows_v[bt, pl.ds(L, L)]
            for r in range(NPW):
                buf[k, r, pl.ds(0, L)] = d0
                buf[k, r, pl.ds(L, L)] = d1
                buf[k, r, pl.ds(2 * L, L)] = t0
                buf[k, r, pl.ds(3 * L, L)] = t1
            return carry
        lax.fori_loop(0, K, fill_k, 0)
        pltpu.async_copy(buf, out_hbm.at[pl.ds(bt0, K), pl.ds(n0, NPW), :], sem)

    def wait_prev(buf, sem):
        pltpu.make_async_copy(
            buf, out_hbm.at[pl.ds(0, K), pl.ds(n0, NPW), :], sem).wait()

    for g in gathers:
        g.wait()

    for j in range(NBUF):
        fill_and_send(j * K, bufs[j], sems[j])

    def body(i, carry):
        bt0 = i * NBUF * K
        for j in range(NBUF):
            wait_prev(bufs[j], sems[j])
            fill_and_send(bt0 + j * K, bufs[j], sems[j])
        return carry

    lax.fori_loop(1, BT // (NBUF * K), body, 0)
    for j in range(NBUF):
        wait_prev(bufs[j], sems[j])


def kernel(daytime, W_day, W_time, W_node):
    dt = daytime.astype(jnp.int32)
    idx = jnp.concatenate(
        [dt[..., 0].reshape(N_CHUNKS, IDX_CHUNK),
         dt[..., 1].reshape(N_CHUNKS, IDX_CHUNK)], axis=0)

    mesh = plsc.VectorSubcoreMesh(core_axis_name="c", subcore_axis_name="s",
                                  num_cores=NC, num_subcores=NS)
    out = pl.kernel(
        _sc_body,
        out_type=jax.ShapeDtypeStruct((BT, N, C), jnp.float32),
        mesh=mesh,
        compiler_params=pltpu.CompilerParams(use_tc_tiling_on_sc=False),
        scratch_types=[
            pltpu.VMEM((2 * N_CHUNKS, IDX_CHUNK), jnp.int32),
            pltpu.VMEM((BT, DAY_SIZE), jnp.float32),
            pltpu.VMEM((BT, TIME_SIZE), jnp.float32),
            pltpu.VMEM((NPW, NODE_SIZE), jnp.float32),
            pltpu.VMEM((K, NPW, C), jnp.float32),
            pltpu.VMEM((K, NPW, C), jnp.float32),
            pltpu.VMEM((K, NPW, C), jnp.float32),
            pltpu.VMEM((K, NPW, C), jnp.float32),
            pltpu.SemaphoreType.DMA,
            pltpu.SemaphoreType.DMA,
            pltpu.SemaphoreType.DMA,
            pltpu.SemaphoreType.DMA,
            pltpu.SemaphoreType.DMA,
        ],
    )(idx, W_day, W_time, W_node)
    return out.reshape(B, T, N, C)


# speedup vs baseline: 1.2403x; 1.0174x over previous
import jax
import jax.numpy as jnp
from jax import lax
from jax.experimental import pallas as pl
from jax.experimental.pallas import tpu as pltpu
from jax.experimental.pallas import tpu_sc as plsc

NC, NS, L = 2, 16, 16
NW = NC * NS
B, T, N = 32, 12, 1024
BT = B * T
DAY_SIZE, TIME_SIZE, NODE_SIZE = 32, 32, 64
C = DAY_SIZE + TIME_SIZE + NODE_SIZE
NPW = N // NW
IDX_CHUNK = 128
N_CHUNKS = BT // IDX_CHUNK
K = 6
NBUF = 4


def _sc_body(idx_hbm, wday_hbm, wtime_hbm, wnode_hbm, out_hbm,
             idx_v, drows_v, trows_v, node_v,
             buf0, buf1, buf2, buf3, sem_g, sem0, sem1, sem2, sem3):
    bufs = (buf0, buf1, buf2, buf3)
    sems = (sem0, sem1, sem2, sem3)
    cid = lax.axis_index("c")
    sid = lax.axis_index("s")
    wid = sid * NC + cid
    n0 = wid * NPW

    pltpu.sync_copy(idx_hbm, idx_v)

    gathers = []
    for j in range(N_CHUNKS):
        gathers.append(pltpu.async_copy(
            wday_hbm.at[idx_v.at[j]],
            drows_v.at[pl.ds(j * IDX_CHUNK, IDX_CHUNK), :], sem_g))
        gathers.append(pltpu.async_copy(
            wtime_hbm.at[idx_v.at[N_CHUNKS + j]],
            trows_v.at[pl.ds(j * IDX_CHUNK, IDX_CHUNK), :], sem_g))

    pltpu.sync_copy(wnode_hbm.at[pl.ds(n0, NPW)], node_v)
    for buf in bufs:
        def init_k(k, carry):
            for r in range(NPW):
                for j in range(NODE_SIZE // L):
                    buf[k, r, pl.ds(NODE_SIZE + j * L, L)] = \
                        node_v[r, pl.ds(j * L, L)]
            return carry
        lax.fori_loop(0, K, init_k, 0)

    def fill_and_send(bt0, buf, sem):
        def fill_k(k, carry):
            bt = bt0 + k
            d0 = drows_v[bt, pl.ds(0, L)]
            d1 = drows_v[bt, pl.ds(L, L)]
            t0 = trows_v[bt, pl.ds(0, L)]
            t1 = trows_v[bt, pl.ds(L, L)]
            for r in range(NPW):
                buf[k, r, pl.ds(0, L)] = d0
                buf[k, r, pl.ds(L, L)] = d1
                buf[k, r, pl.ds(2 * L, L)] = t0
                buf[k, r, pl.ds(3 * L, L)] = t1
            return carry
        lax.fori_loop(0, K, fill_k, 0)
        pltpu.async_copy(buf, out_hbm.at[pl.ds(bt0, K), pl.ds(n0, NPW), :], sem)

    def wait_prev(buf, sem):
        pltpu.make_async_copy(
            buf, out_hbm.at[pl.ds(0, K), pl.ds(n0, NPW), :], sem).wait()

    for g in gathers:
        g.wait()

    for j in range(NBUF):
        fill_and_send(j * K, bufs[j], sems[j])

    def body(i, carry):
        bt0 = i * NBUF * K
        for j in range(NBUF):
            wait_prev(bufs[j], sems[j])
            fill_and_send(bt0 + j * K, bufs[j], sems[j])
        return carry

    lax.fori_loop(1, BT // (NBUF * K), body, 0)
    for j in range(NBUF):
        wait_prev(bufs[j], sems[j])


def kernel(daytime, W_day, W_time, W_node):
    dt = daytime.astype(jnp.int32)
    idx = jnp.concatenate(
        [dt[..., 0].reshape(N_CHUNKS, IDX_CHUNK),
         dt[..., 1].reshape(N_CHUNKS, IDX_CHUNK)], axis=0)

    mesh = plsc.VectorSubcoreMesh(core_axis_name="c", subcore_axis_name="s",
                                  num_cores=NC, num_subcores=NS)
    out = pl.kernel(
        _sc_body,
        out_type=jax.ShapeDtypeStruct((BT, N, C), jnp.float32),
        mesh=mesh,
        compiler_params=pltpu.CompilerParams(use_tc_tiling_on_sc=False),
        scratch_types=[
            pltpu.VMEM((2 * N_CHUNKS, IDX_CHUNK), jnp.int32),
            pltpu.VMEM((BT, DAY_SIZE), jnp.float32),
            pltpu.VMEM((BT, TIME_SIZE), jnp.float32),
            pltpu.VMEM((NPW, NODE_SIZE), jnp.float32),
            pltpu.VMEM((K, NPW, C), jnp.float32),
            pltpu.VMEM((K, NPW, C), jnp.float32),
            pltpu.VMEM((K, NPW, C), jnp.float32),
            pltpu.VMEM((K, NPW, C), jnp.float32),
            pltpu.SemaphoreType.DMA,
            pltpu.SemaphoreType.DMA,
            pltpu.SemaphoreType.DMA,
            pltpu.SemaphoreType.DMA,
            pltpu.SemaphoreType.DMA,
        ],
    )(idx, W_day, W_time, W_node)
    return out.reshape(B, T, N, C)
